# baseline (speedup 1.0000x reference)
import functools

import jax
import jax.numpy as jnp
from jax import lax
from jax.experimental import pallas as pl
from jax.experimental.pallas import tpu as pltpu
from jax.experimental.pallas import tpu_sc as plsc

_NC = 2
_NS = 16
_NW = _NC * _NS


def _sc_position_add(x, input_pos, emb_weight):
    B, S, D = x.shape
    PB = 4
    NSBLK = _NW // (B // PB)
    SPW = S // NSBLK
    CH = 1
    NCHUNK = SPW // CH
    NBUF = 8
    NG = NCHUNK // NBUF
    LANES = 16

    mesh = plsc.VectorSubcoreMesh(core_axis_name="c", subcore_axis_name="s")

    @functools.partial(
        pl.kernel,
        mesh=mesh,
        out_type=jax.ShapeDtypeStruct((B, S, D), jnp.float32),
        scratch_types=[
            pltpu.VMEM((NCHUNK, CH), jnp.int32),
            pltpu.VMEM((NBUF, PB, CH, D), jnp.float32),
            pltpu.VMEM((NBUF, CH, D), jnp.float32),
            [pltpu.SemaphoreType.DMA] * NBUF,
            [pltpu.SemaphoreType.DMA] * NBUF,
            [pltpu.SemaphoreType.DMA] * NBUF,
        ],
    )
    def body(x_hbm, pos_hbm, emb_hbm, out_hbm, idx_all, xbuf, ebuf,
             sx, se, so):
        wid = lax.axis_index("s") * _NC + lax.axis_index("c")
        bp = wid // NSBLK
        sblk = wid % NSBLK
        b0 = PB * bp
        s_base = sblk * SPW

        pltpu.sync_copy(pos_hbm.at[pl.ds(sblk * NCHUNK, NCHUNK), :], idx_all)

        def x_copy(i, b):
            s0 = s_base + i * CH
            return pltpu.make_async_copy(
                x_hbm.at[pl.ds(b0, PB), pl.ds(s0, CH), :], xbuf.at[b], sx[b])

        def e_copy(i, b):
            return pltpu.make_async_copy(
                emb_hbm.at[idx_all.at[i]], ebuf.at[b], se[b])

        def o_copy(i, b):
            s0 = s_base + i * CH
            return pltpu.make_async_copy(
                xbuf.at[b], out_hbm.at[pl.ds(b0, PB), pl.ds(s0, CH), :], so[b])

        def issue_loads(i, b):
            x_copy(i, b).start()
            e_copy(i, b).start()

        def add_chunk(b):
            for r in range(CH):
                @plsc.parallel_loop(0, D // LANES, unroll=8)
                def _(k, _r=r, _b=b):
                    off = k * LANES
                    e = ebuf[_b, _r, pl.ds(off, LANES)]
                    for j in range(PB):
                        plsc.addupdate(xbuf.at[_b, j, _r, pl.ds(off, LANES)], e)

        LOOKAHEAD = 6

        for b in range(LOOKAHEAD):
            issue_loads(b, b)

        for i in range(NBUF):
            b = i % NBUF
            x_copy(i, b).wait()
            e_copy(i, b).wait()
            add_chunk(b)
            o_copy(i, b).start()
            nb = (b + LOOKAHEAD) % NBUF
            if i >= NBUF - LOOKAHEAD:
                o_copy(i - (NBUF - LOOKAHEAD), nb).wait()
            issue_loads(i + LOOKAHEAD, nb)

        def group(g, carry):
            for b in range(NBUF):
                i = g * NBUF + b
                x_copy(i, b).wait()
                e_copy(i, b).wait()
                add_chunk(b)
                o_copy(i, b).start()
                nb = (b + LOOKAHEAD) % NBUF
                o_copy(i - (NBUF - LOOKAHEAD), nb).wait()

                @pl.when(i + LOOKAHEAD < NCHUNK)
                def _():
                    issue_loads(i + LOOKAHEAD, nb)
            return carry

        lax.fori_loop(1, NG, group, 0)

        for i in range(NCHUNK - (NBUF - LOOKAHEAD), NCHUNK):
            o_copy(i, i % NBUF).wait()

    pos2 = input_pos.reshape(S // CH, CH)
    return body(x, pos2, emb_weight)


def kernel(x, input_pos, emb_weight):
    return _sc_position_add(x, input_pos, emb_weight)


# speedup vs baseline: 1.0022x; 1.0022x over previous
import functools

import jax
import jax.numpy as jnp
from jax import lax
from jax.experimental import pallas as pl
from jax.experimental.pallas import tpu as pltpu
from jax.experimental.pallas import tpu_sc as plsc

_NC = 2
_NS = 16
_NW = _NC * _NS


def _sc_position_add(x, input_pos, emb_weight):
    B, S, D = x.shape
    PB = 4
    NSBLK = _NW // (B // PB)
    SPW = S // NSBLK
    CH = 2
    NCHUNK = SPW // CH
    NBUF = 4
    NG = NCHUNK // NBUF
    LANES = 16

    mesh = plsc.VectorSubcoreMesh(core_axis_name="c", subcore_axis_name="s")

    @functools.partial(
        pl.kernel,
        mesh=mesh,
        out_type=jax.ShapeDtypeStruct((B, S, D), jnp.float32),
        scratch_types=[
            pltpu.VMEM((NCHUNK, CH), jnp.int32),
            pltpu.VMEM((NBUF, PB, CH, D), jnp.float32),
            pltpu.VMEM((NBUF, CH, D), jnp.float32),
            [pltpu.SemaphoreType.DMA] * NBUF,
            [pltpu.SemaphoreType.DMA] * NBUF,
            [pltpu.SemaphoreType.DMA] * NBUF,
        ],
    )
    def body(x_hbm, pos_hbm, emb_hbm, out_hbm, idx_all, xbuf, ebuf,
             sx, se, so):
        wid = lax.axis_index("s") * _NC + lax.axis_index("c")
        bp = wid // NSBLK
        sblk = wid % NSBLK
        b0 = PB * bp
        s_base = sblk * SPW

        pltpu.sync_copy(pos_hbm.at[pl.ds(sblk * NCHUNK, NCHUNK), :], idx_all)

        def x_copy(i, b):
            s0 = s_base + i * CH
            return pltpu.make_async_copy(
                x_hbm.at[pl.ds(b0, PB), pl.ds(s0, CH), :], xbuf.at[b], sx[b])

        def e_copy(i, b):
            return pltpu.make_async_copy(
                emb_hbm.at[idx_all.at[i]], ebuf.at[b], se[b])

        def o_copy(i, b):
            s0 = s_base + i * CH
            return pltpu.make_async_copy(
                xbuf.at[b], out_hbm.at[pl.ds(b0, PB), pl.ds(s0, CH), :], so[b])

        def issue_loads(i, b):
            x_copy(i, b).start()
            e_copy(i, b).start()

        def add_chunk(b):
            for r in range(CH):
                @plsc.parallel_loop(0, D // LANES, unroll=8)
                def _(k, _r=r, _b=b):
                    off = k * LANES
                    e = ebuf[_b, _r, pl.ds(off, LANES)]
                    for j in range(PB):
                        plsc.addupdate(xbuf.at[_b, j, _r, pl.ds(off, LANES)], e)

        LOOKAHEAD = 3

        for b in range(LOOKAHEAD):
            issue_loads(b, b)

        for i in range(NBUF):
            b = i % NBUF
            x_copy(i, b).wait()
            e_copy(i, b).wait()
            add_chunk(b)
            o_copy(i, b).start()
            nb = (b + LOOKAHEAD) % NBUF
            if i >= NBUF - LOOKAHEAD:
                o_copy(i - (NBUF - LOOKAHEAD), nb).wait()
            issue_loads(i + LOOKAHEAD, nb)

        def group(g, carry):
            for b in range(NBUF):
                i = g * NBUF + b
                x_copy(i, b).wait()
                e_copy(i, b).wait()
                add_chunk(b)
                o_copy(i, b).start()
                nb = (b + LOOKAHEAD) % NBUF
                o_copy(i - (NBUF - LOOKAHEAD), nb).wait()

                @pl.when(i + LOOKAHEAD < NCHUNK)
                def _():
                    issue_loads(i + LOOKAHEAD, nb)
            return carry

        lax.fori_loop(1, NG, group, 0)

        for i in range(NCHUNK - (NBUF - LOOKAHEAD), NCHUNK):
            o_copy(i, i % NBUF).wait()

    pos2 = input_pos.reshape(S // CH, CH)
    return body(x, pos2, emb_weight)


def kernel(x, input_pos, emb_weight):
    return _sc_position_add(x, input_pos, emb_weight)
